# Initial kernel scaffold; baseline (speedup 1.0000x reference)
import functools

import jax
import jax.numpy as jnp
from jax import lax
from jax.experimental import pallas as pl
from jax.experimental.pallas import tpu as pltpu
from jax.experimental.pallas import tpu_sc as plsc

N = 10000
E = 320000
M = 128
NCRYS = 1024

_NC = 2
_NS = 16
_NW = _NC * _NS
_GCH = 128


def _leaky(x):
    return jnp.where(x >= 0, x, 0.2 * x)


def _sc_gather_body(table_hbm, idx_hbm, out_hbm, idx_v, rows_v, sem):
    wid = lax.axis_index("s") * _NC + lax.axis_index("c")
    nchunks = idx_hbm.shape[0] // (_NW * _GCH)
    base = wid * (nchunks * _GCH)

    def chunk(c, _):
        off = base + c * _GCH
        pltpu.sync_copy(idx_hbm.at[pl.ds(off, _GCH)], idx_v)
        pltpu.async_copy(table_hbm.at[idx_v], rows_v, sem).wait()
        pltpu.sync_copy(rows_v, out_hbm.at[pl.ds(off, _GCH)])
        return 0

    lax.fori_loop(0, nchunks, chunk, 0)


def _sc_gather(table, idx_padded):
    ep = idx_padded.shape[0]
    kfn = pl.kernel(
        _sc_gather_body,
        out_type=jax.ShapeDtypeStruct((ep, M), jnp.float32),
        mesh=plsc.VectorSubcoreMesh(core_axis_name="c", subcore_axis_name="s"),
        scratch_types=[
            pltpu.VMEM((_GCH,), jnp.int32),
            pltpu.VMEM((_GCH, M), jnp.float32),
            pltpu.SemaphoreType.DMA,
        ],
    )
    return kfn(table, idx_padded)


def _sc_scatter_body(vals_hbm, idx_hbm, zeros_hbm, out_hbm,
                     idx_v, rows_v, idx_t, rows_t, accum, sem):
    cid = lax.axis_index("c")
    sid = lax.axis_index("s")
    wid = sid * _NC + cid
    per_w = vals_hbm.shape[0] // _NW
    nfull = per_w // _GCH
    tail = per_w - nfull * _GCH
    base = wid * per_w

    @pl.when(sid == 0)
    def _init():
        pltpu.sync_copy(zeros_hbm, accum)

    plsc.subcore_barrier()

    def chunk(c, _):
        off = base + c * _GCH
        pltpu.sync_copy(idx_hbm.at[pl.ds(off, _GCH)], idx_v)
        pltpu.sync_copy(vals_hbm.at[pl.ds(off, _GCH)], rows_v)
        pltpu.sync_copy(rows_v, accum.at[idx_v], add=True)
        return 0

    lax.fori_loop(0, nfull, chunk, 0)

    if tail:
        toff = base + nfull * _GCH
        pltpu.sync_copy(idx_hbm.at[pl.ds(toff, tail)], idx_t)
        pltpu.sync_copy(vals_hbm.at[pl.ds(toff, tail)], rows_t)
        pltpu.sync_copy(rows_t, accum.at[idx_t], add=True)

    plsc.subcore_barrier()

    stripe = accum.shape[0] // _NS
    pltpu.sync_copy(
        accum.at[pl.ds(sid * stripe, stripe)],
        out_hbm.at[pl.ds(cid * accum.shape[0] + sid * stripe, stripe)])


def _sc_scatter(vals, idx, zeros_nm):
    per_w = vals.shape[0] // _NW
    tail = per_w - (per_w // _GCH) * _GCH
    kfn = pl.kernel(
        _sc_scatter_body,
        out_type=jax.ShapeDtypeStruct((2 * N, M), jnp.float32),
        mesh=plsc.VectorSubcoreMesh(core_axis_name="c", subcore_axis_name="s"),
        scratch_types=[
            pltpu.VMEM((_GCH,), jnp.int32),
            pltpu.VMEM((_GCH, M), jnp.float32),
            pltpu.VMEM((max(tail, 8),), jnp.int32),
            pltpu.VMEM((max(tail, 8), M), jnp.float32),
            pltpu.VMEM_SHARED((N, M), jnp.float32),
            pltpu.SemaphoreType.DMA,
        ],
    )
    return kfn(vals, idx, zeros_nm)


def _embed_body(x_ref, w_ref, b_ref, o_ref):
    o_ref[...] = (
        jnp.dot(x_ref[...], w_ref[...], preferred_element_type=jnp.float32)
        + b_ref[...])


def _embed(x, w_t, b_row, block_rows):
    n, k = x.shape
    m = w_t.shape[1]
    grid = n // block_rows
    return pl.pallas_call(
        _embed_body,
        grid=(grid,),
        in_specs=[
            pl.BlockSpec((block_rows, k), lambda i: (i, 0)),
            pl.BlockSpec((k, m), lambda i: (0, 0)),
            pl.BlockSpec((1, m), lambda i: (0, 0)),
        ],
        out_specs=pl.BlockSpec((block_rows, m), lambda i: (i, 0)),
        out_shape=jax.ShapeDtypeStruct((n, m), jnp.float32),
    )(x, w_t, b_row)


def _edge_body(g1, g2, nbr, w1a, w1b, w1c, b1, w2, b2, w3, b3, ek_o, nbr_o):
    t = jnp.dot(g1[...], w1a[...], preferred_element_type=jnp.float32)
    t += jnp.dot(g2[...], w1b[...], preferred_element_type=jnp.float32)
    t += jnp.dot(nbr[...], w1c[...], preferred_element_type=jnp.float32)
    h = _leaky(t + b1[...])
    h = _leaky(jnp.dot(h, w2[...], preferred_element_type=jnp.float32)
               + b2[...])
    ek = jnp.dot(h, w3[...], preferred_element_type=jnp.float32) + b3[...]
    ek_o[...] = ek
    nbr_o[...] = nbr[...] + ek


def _edge_mlp(g1, g2, nbr, w1a, w1b, w1c, b1, w2, b2, w3, b3, block_rows):
    grid = E // block_rows
    row = lambda i: (i, 0)
    fix = lambda i: (0, 0)
    return pl.pallas_call(
        _edge_body,
        grid=(grid,),
        in_specs=[
            pl.BlockSpec((block_rows, M), row),
            pl.BlockSpec((block_rows, M), row),
            pl.BlockSpec((block_rows, M), row),
            pl.BlockSpec((M, M), fix), pl.BlockSpec((M, M), fix),
            pl.BlockSpec((M, M), fix), pl.BlockSpec((1, M), fix),
            pl.BlockSpec((M, M), fix), pl.BlockSpec((1, M), fix),
            pl.BlockSpec((M, M), fix), pl.BlockSpec((1, M), fix),
        ],
        out_specs=[
            pl.BlockSpec((block_rows, M), row),
            pl.BlockSpec((block_rows, M), row),
        ],
        out_shape=[
            jax.ShapeDtypeStruct((E, M), jnp.float32),
            jax.ShapeDtypeStruct((E, M), jnp.float32),
        ],
    )(g1, g2, nbr, w1a, w1b, w1c, b1, w2, b2, w3, b3)


def _node_body(atom, p0, p1, nn, wa, wr, b1, w2, b2, w3, b3, g, bb, out):
    rho = (p0[...] + p1[...]) / nn[...]
    t = jnp.dot(atom[...], wa[...], preferred_element_type=jnp.float32)
    t += jnp.dot(rho, wr[...], preferred_element_type=jnp.float32)
    h = _leaky(t + b1[...])
    h = _leaky(jnp.dot(h, w2[...], preferred_element_type=jnp.float32)
               + b2[...])
    vi = jnp.dot(h, w3[...], preferred_element_type=jnp.float32) + b3[...]
    mu = jnp.mean(vi, axis=0, keepdims=True)
    var = jnp.mean((vi - mu) ** 2, axis=0, keepdims=True)
    vi = (vi - mu) / jnp.sqrt(var + 1e-5) * g[...] + bb[...]
    out[...] = atom[...] + vi


def _node_mlp(atom, p0, p1, nn_col, wa, wr, b1, w2, b2, w3, b3, g_row, b_row):
    return pl.pallas_call(
        _node_body,
        out_shape=jax.ShapeDtypeStruct((N, M), jnp.float32),
    )(atom, p0, p1, nn_col, wa, wr, b1, w2, b2, w3, b3, g_row, b_row)


def _pool_body(vi, p0, p1, nn, cidx, ua, ue, ub1, uw2, ub2,
               fcw, fcb, fc1w, fc1b, ow, ob, out,
               gfa, gfb, cnt):
    i = pl.program_id(0)
    nblk = pl.num_programs(0)
    rows = vi.shape[0]

    @pl.when(i == 0)
    def _zero():
        gfa[...] = jnp.zeros_like(gfa)
        gfb[...] = jnp.zeros_like(gfb)
        cnt[...] = jnp.zeros_like(cnt)

    eks = (p0[...] + p1[...]) / nn[...]
    iota = lax.broadcasted_iota(jnp.int32, (rows, NCRYS), 1)
    onehot = (iota == cidx[...]).astype(jnp.float32)
    dn = (((0,), (0,)), ((), ()))
    gfa[...] += lax.dot_general(onehot, vi[...], dn,
                                preferred_element_type=jnp.float32)
    gfb[...] += lax.dot_general(onehot, eks, dn,
                                preferred_element_type=jnp.float32)
    cnt[...] += lax.dot_general(onehot, jnp.ones((rows, M), jnp.float32), dn,
                                preferred_element_type=jnp.float32)

    @pl.when(i == nblk - 1)
    def _head():
        pa = gfa[...] / cnt[...]
        pb = gfb[...] / cnt[...]
        z = jnp.dot(pa, ua[...], preferred_element_type=jnp.float32)
        z += jnp.dot(pb, ue[...], preferred_element_type=jnp.float32)
        z = _leaky(z + ub1[...])
        z = jnp.tanh(jnp.dot(z, uw2[...], preferred_element_type=jnp.float32)
                     + ub2[...])
        c = _leaky(jnp.dot(z, fcw[...], preferred_element_type=jnp.float32)
                   + fcb[...])
        c = _leaky(jnp.dot(c, fc1w[...], preferred_element_type=jnp.float32)
                   + fc1b[...])
        out[...] = (jnp.dot(c, ow[...], preferred_element_type=jnp.float32)
                    + ob[...])


def _pool_head(vi, p0, p1, nn_col, cidx_col, ua, ue, ub1, uw2, ub2,
               fcw, fcb, fc1w, fc1b, ow, ob, block_rows):
    grid = N // block_rows
    row = lambda i: (i, 0)
    fix = lambda i: (0, 0)
    return pl.pallas_call(
        _pool_body,
        grid=(grid,),
        in_specs=[
            pl.BlockSpec((block_rows, M), row),
            pl.BlockSpec((block_rows, M), row),
            pl.BlockSpec((block_rows, M), row),
            pl.BlockSpec((block_rows, 1), row),
            pl.BlockSpec((block_rows, 1), row),
            pl.BlockSpec((M, M), fix), pl.BlockSpec((M, M), fix),
            pl.BlockSpec((1, M), fix),
            pl.BlockSpec((M, M), fix), pl.BlockSpec((1, M), fix),
            pl.BlockSpec((M, M), fix), pl.BlockSpec((1, M), fix),
            pl.BlockSpec((M, M), fix), pl.BlockSpec((1, M), fix),
            pl.BlockSpec((M, 1), fix), pl.BlockSpec((1, 1), fix),
        ],
        out_specs=pl.BlockSpec((NCRYS, 1), fix),
        out_shape=jax.ShapeDtypeStruct((NCRYS, 1), jnp.float32),
        scratch_shapes=[
            pltpu.VMEM((NCRYS, M), jnp.float32),
            pltpu.VMEM((NCRYS, M), jnp.float32),
            pltpu.VMEM((NCRYS, M), jnp.float32),
        ],
    )(vi, p0, p1, nn_col, cidx_col, ua, ue, ub1, uw2, ub2,
      fcw, fcb, fc1w, fc1b, ow, ob)


def kernel(atom_fea, nbr_fea, nbr_fea_idx1, nbr_fea_idx2, num_nbrs,
           crystal_atom_idx, params):
    p = params
    rowb = lambda b: b.reshape(1, -1)

    idx_all = jnp.concatenate([nbr_fea_idx1, nbr_fea_idx2])
    ep = ((2 * E + _NW * _GCH - 1) // (_NW * _GCH)) * (_NW * _GCH)
    idx_all = jnp.pad(idx_all, (0, ep - 2 * E))

    zeros_nm = jnp.zeros((N, M), jnp.float32)
    nn_col = num_nbrs.reshape(N, 1)
    cidx_col = crystal_atom_idx.reshape(N, 1)

    atom = _embed(atom_fea, p["node_W"].T, rowb(p["node_b"]), 2000)
    nbr = _embed(nbr_fea, p["edge_W"].T, rowb(p["edge_b"]), 2000)

    eks_parts = None
    nconv = len(p["convs"])
    for li, c in enumerate(p["convs"]):
        gath = _sc_gather(atom, idx_all)
        e_w1t = c["eW1"].T
        ek, nbr = _edge_mlp(
            gath[:E], gath[E:2 * E], nbr,
            e_w1t[:M], e_w1t[M:2 * M], e_w1t[2 * M:], rowb(c["eb1"]),
            c["eW2"].T, rowb(c["eb2"]), c["eW3"].T, rowb(c["eb3"]), 2000)
        rho_parts = _sc_scatter(ek, nbr_fea_idx1, zeros_nm)
        v_w1t = c["vW1"].T
        atom = _node_mlp(
            atom, rho_parts[:N], rho_parts[N:], nn_col,
            v_w1t[:M], v_w1t[M:], rowb(c["vb1"]),
            c["vW2"].T, rowb(c["vb2"]), c["vW3"].T, rowb(c["vb3"]),
            rowb(c["bn_g"]), rowb(c["bn_b"]))
        if li == nconv - 1:
            eks_parts = _sc_scatter(nbr, nbr_fea_idx1, zeros_nm)

    u_w1t = p["uW1"].T
    return _pool_head(
        atom, eks_parts[:N], eks_parts[N:], nn_col, cidx_col,
        u_w1t[:M], u_w1t[M:], rowb(p["ub1"]),
        p["uW2"].T, rowb(p["ub2"]),
        p["fcW"].T, rowb(p["fcb"]),
        p["fc1W"].T, rowb(p["fc1b"]),
        p["outW"].T, rowb(p["outb"]), 2000)


# speedup vs baseline: 1.4031x; 1.4031x over previous
import functools

import jax
import jax.numpy as jnp
from jax import lax
from jax.experimental import pallas as pl
from jax.experimental.pallas import tpu as pltpu
from jax.experimental.pallas import tpu_sc as plsc

N = 10000
E = 320000
M = 128
NCRYS = 1024

_NC = 2
_NS = 16
_NW = _NC * _NS
_GCH = 128


def _leaky(x):
    return jnp.where(x >= 0, x, 0.2 * x)


def _sc_gather_body(table_hbm, idx_hbm, out_hbm, idx_v, rows_v, sem):
    wid = lax.axis_index("s") * _NC + lax.axis_index("c")
    nchunks = idx_hbm.shape[0] // (_NW * _GCH)
    base = wid * (nchunks * _GCH)

    def chunk(c, _):
        off = base + c * _GCH
        pltpu.sync_copy(idx_hbm.at[pl.ds(off, _GCH)], idx_v)
        pltpu.async_copy(table_hbm.at[idx_v], rows_v, sem).wait()
        pltpu.sync_copy(rows_v, out_hbm.at[pl.ds(off, _GCH)])
        return 0

    lax.fori_loop(0, nchunks, chunk, 0)


def _sc_gather(table, idx_padded):
    ep = idx_padded.shape[0]
    kfn = pl.kernel(
        _sc_gather_body,
        out_type=jax.ShapeDtypeStruct((ep, M), jnp.float32),
        mesh=plsc.VectorSubcoreMesh(core_axis_name="c", subcore_axis_name="s"),
        scratch_types=[
            pltpu.VMEM((_GCH,), jnp.int32),
            pltpu.VMEM((_GCH, M), jnp.float32),
            pltpu.SemaphoreType.DMA,
        ],
    )
    return kfn(table, idx_padded)


def _sc_scatter_body(vals_hbm, idx_hbm, zeros_hbm, out_hbm,
                     idx_v, rows_v, idx_t, rows_t, accum, sem):
    cid = lax.axis_index("c")
    sid = lax.axis_index("s")
    wid = sid * _NC + cid
    per_w = vals_hbm.shape[0] // _NW
    nfull = per_w // _GCH
    tail = per_w - nfull * _GCH
    base = wid * per_w

    @pl.when(sid == 0)
    def _init():
        pltpu.sync_copy(zeros_hbm, accum)

    plsc.subcore_barrier()

    def chunk(c, _):
        off = base + c * _GCH
        pltpu.sync_copy(idx_hbm.at[pl.ds(off, _GCH)], idx_v)
        pltpu.sync_copy(vals_hbm.at[pl.ds(off, _GCH)], rows_v)
        pltpu.sync_copy(rows_v, accum.at[idx_v], add=True)
        return 0

    lax.fori_loop(0, nfull, chunk, 0)

    if tail:
        toff = base + nfull * _GCH
        pltpu.sync_copy(idx_hbm.at[pl.ds(toff, tail)], idx_t)
        pltpu.sync_copy(vals_hbm.at[pl.ds(toff, tail)], rows_t)
        pltpu.sync_copy(rows_t, accum.at[idx_t], add=True)

    plsc.subcore_barrier()

    stripe = (accum.shape[0] // _NS) // 8 * 8
    last = accum.shape[0] - stripe * (_NS - 1)

    @pl.when(sid < _NS - 1)
    def _dump_main():
        pltpu.sync_copy(
            accum.at[pl.ds(sid * stripe, stripe)],
            out_hbm.at[pl.ds(cid * accum.shape[0] + sid * stripe, stripe)])

    @pl.when(sid == _NS - 1)
    def _dump_last():
        pltpu.sync_copy(
            accum.at[pl.ds(stripe * (_NS - 1), last)],
            out_hbm.at[pl.ds(cid * accum.shape[0] + stripe * (_NS - 1), last)])


def _sc_scatter(vals, idx, zeros_nm):
    per_w = vals.shape[0] // _NW
    tail = per_w - (per_w // _GCH) * _GCH
    kfn = pl.kernel(
        _sc_scatter_body,
        out_type=jax.ShapeDtypeStruct((2 * N, M), jnp.float32),
        mesh=plsc.VectorSubcoreMesh(core_axis_name="c", subcore_axis_name="s"),
        scratch_types=[
            pltpu.VMEM((_GCH,), jnp.int32),
            pltpu.VMEM((_GCH, M), jnp.float32),
            pltpu.VMEM((max(tail, 8),), jnp.int32),
            pltpu.VMEM((max(tail, 8), M), jnp.float32),
            pltpu.VMEM_SHARED((N, M), jnp.float32),
            pltpu.SemaphoreType.DMA,
        ],
    )
    return kfn(vals, idx, zeros_nm)


def _embed_body(x_ref, w_ref, b_ref, o_ref):
    o_ref[...] = (
        jnp.dot(x_ref[...], w_ref[...], preferred_element_type=jnp.float32, precision=lax.Precision.HIGHEST)
        + b_ref[...])


def _embed(x, w_t, b_row, block_rows):
    n, k = x.shape
    m = w_t.shape[1]
    grid = n // block_rows
    return pl.pallas_call(
        _embed_body,
        grid=(grid,),
        in_specs=[
            pl.BlockSpec((block_rows, k), lambda i: (i, 0)),
            pl.BlockSpec((k, m), lambda i: (0, 0)),
            pl.BlockSpec((1, m), lambda i: (0, 0)),
        ],
        out_specs=pl.BlockSpec((block_rows, m), lambda i: (i, 0)),
        out_shape=jax.ShapeDtypeStruct((n, m), jnp.float32),
    )(x, w_t, b_row)


def _edge_body(g1, g2, nbr, w1a, w1b, w1c, b1, w2, b2, w3, b3, ek_o, nbr_o):
    t = jnp.dot(g1[...], w1a[...], preferred_element_type=jnp.float32, precision=lax.Precision.HIGHEST)
    t += jnp.dot(g2[...], w1b[...], preferred_element_type=jnp.float32, precision=lax.Precision.HIGHEST)
    t += jnp.dot(nbr[...], w1c[...], preferred_element_type=jnp.float32, precision=lax.Precision.HIGHEST)
    h = _leaky(t + b1[...])
    h = _leaky(jnp.dot(h, w2[...], preferred_element_type=jnp.float32, precision=lax.Precision.HIGHEST)
               + b2[...])
    ek = jnp.dot(h, w3[...], preferred_element_type=jnp.float32, precision=lax.Precision.HIGHEST) + b3[...]
    ek_o[...] = ek
    nbr_o[...] = nbr[...] + ek


def _edge_mlp(g1, g2, nbr, w1a, w1b, w1c, b1, w2, b2, w3, b3, block_rows):
    grid = E // block_rows
    row = lambda i: (i, 0)
    fix = lambda i: (0, 0)
    return pl.pallas_call(
        _edge_body,
        grid=(grid,),
        in_specs=[
            pl.BlockSpec((block_rows, M), row),
            pl.BlockSpec((block_rows, M), row),
            pl.BlockSpec((block_rows, M), row),
            pl.BlockSpec((M, M), fix), pl.BlockSpec((M, M), fix),
            pl.BlockSpec((M, M), fix), pl.BlockSpec((1, M), fix),
            pl.BlockSpec((M, M), fix), pl.BlockSpec((1, M), fix),
            pl.BlockSpec((M, M), fix), pl.BlockSpec((1, M), fix),
        ],
        out_specs=[
            pl.BlockSpec((block_rows, M), row),
            pl.BlockSpec((block_rows, M), row),
        ],
        out_shape=[
            jax.ShapeDtypeStruct((E, M), jnp.float32),
            jax.ShapeDtypeStruct((E, M), jnp.float32),
        ],
    )(g1, g2, nbr, w1a, w1b, w1c, b1, w2, b2, w3, b3)


def _node_body(atom, p0, p1, nn, wa, wr, b1, w2, b2, w3, b3, g, bb, out):
    rho = (p0[...] + p1[...]) / nn[...]
    t = jnp.dot(atom[...], wa[...], preferred_element_type=jnp.float32, precision=lax.Precision.HIGHEST)
    t += jnp.dot(rho, wr[...], preferred_element_type=jnp.float32, precision=lax.Precision.HIGHEST)
    h = _leaky(t + b1[...])
    h = _leaky(jnp.dot(h, w2[...], preferred_element_type=jnp.float32, precision=lax.Precision.HIGHEST)
               + b2[...])
    vi = jnp.dot(h, w3[...], preferred_element_type=jnp.float32, precision=lax.Precision.HIGHEST) + b3[...]
    mu = jnp.mean(vi, axis=0, keepdims=True)
    var = jnp.mean((vi - mu) ** 2, axis=0, keepdims=True)
    vi = (vi - mu) / jnp.sqrt(var + 1e-5) * g[...] + bb[...]
    out[...] = atom[...] + vi


def _node_mlp(atom, p0, p1, nn_col, wa, wr, b1, w2, b2, w3, b3, g_row, b_row):
    return pl.pallas_call(
        _node_body,
        out_shape=jax.ShapeDtypeStruct((N, M), jnp.float32),
    )(atom, p0, p1, nn_col, wa, wr, b1, w2, b2, w3, b3, g_row, b_row)


def _pool_body(vi, p0, p1, nn, cidx, ua, ue, ub1, uw2, ub2,
               fcw, fcb, fc1w, fc1b, ow, ob, out,
               gfa, gfb, cnt):
    i = pl.program_id(0)
    nblk = pl.num_programs(0)
    rows = vi.shape[0]

    @pl.when(i == 0)
    def _zero():
        gfa[...] = jnp.zeros_like(gfa)
        gfb[...] = jnp.zeros_like(gfb)
        cnt[...] = jnp.zeros_like(cnt)

    eks = (p0[...] + p1[...]) / nn[...]
    iota = lax.broadcasted_iota(jnp.int32, (rows, NCRYS), 1)
    onehot = (iota == cidx[...]).astype(jnp.float32)
    dn = (((0,), (0,)), ((), ()))
    gfa[...] += lax.dot_general(onehot, vi[...], dn,
                                preferred_element_type=jnp.float32, precision=lax.Precision.HIGHEST)
    gfb[...] += lax.dot_general(onehot, eks, dn,
                                preferred_element_type=jnp.float32, precision=lax.Precision.HIGHEST)
    cnt[...] += lax.dot_general(onehot, jnp.ones((rows, M), jnp.float32), dn,
                                preferred_element_type=jnp.float32, precision=lax.Precision.HIGHEST)

    @pl.when(i == nblk - 1)
    def _head():
        pa = gfa[...] / cnt[...]
        pb = gfb[...] / cnt[...]
        z = jnp.dot(pa, ua[...], preferred_element_type=jnp.float32, precision=lax.Precision.HIGHEST)
        z += jnp.dot(pb, ue[...], preferred_element_type=jnp.float32, precision=lax.Precision.HIGHEST)
        z = _leaky(z + ub1[...])
        z = jnp.tanh(jnp.dot(z, uw2[...], preferred_element_type=jnp.float32, precision=lax.Precision.HIGHEST)
                     + ub2[...])
        c = _leaky(jnp.dot(z, fcw[...], preferred_element_type=jnp.float32, precision=lax.Precision.HIGHEST)
                   + fcb[...])
        c = _leaky(jnp.dot(c, fc1w[...], preferred_element_type=jnp.float32, precision=lax.Precision.HIGHEST)
                   + fc1b[...])
        out[...] = (jnp.dot(c, ow[...], preferred_element_type=jnp.float32, precision=lax.Precision.HIGHEST)
                    + ob[...])


def _pool_head(vi, p0, p1, nn_col, cidx_col, ua, ue, ub1, uw2, ub2,
               fcw, fcb, fc1w, fc1b, ow, ob, block_rows):
    grid = N // block_rows
    row = lambda i: (i, 0)
    fix = lambda i: (0, 0)
    return pl.pallas_call(
        _pool_body,
        grid=(grid,),
        in_specs=[
            pl.BlockSpec((block_rows, M), row),
            pl.BlockSpec((block_rows, M), row),
            pl.BlockSpec((block_rows, M), row),
            pl.BlockSpec((block_rows, 1), row),
            pl.BlockSpec((block_rows, 1), row),
            pl.BlockSpec((M, M), fix), pl.BlockSpec((M, M), fix),
            pl.BlockSpec((1, M), fix),
            pl.BlockSpec((M, M), fix), pl.BlockSpec((1, M), fix),
            pl.BlockSpec((M, M), fix), pl.BlockSpec((1, M), fix),
            pl.BlockSpec((M, M), fix), pl.BlockSpec((1, M), fix),
            pl.BlockSpec((M, 1), fix), pl.BlockSpec((1, 1), fix),
        ],
        out_specs=pl.BlockSpec((NCRYS, 1), fix),
        out_shape=jax.ShapeDtypeStruct((NCRYS, 1), jnp.float32),
        scratch_shapes=[
            pltpu.VMEM((NCRYS, M), jnp.float32),
            pltpu.VMEM((NCRYS, M), jnp.float32),
            pltpu.VMEM((NCRYS, M), jnp.float32),
        ],
    )(vi, p0, p1, nn_col, cidx_col, ua, ue, ub1, uw2, ub2,
      fcw, fcb, fc1w, fc1b, ow, ob)


def kernel(atom_fea, nbr_fea, nbr_fea_idx1, nbr_fea_idx2, num_nbrs,
           crystal_atom_idx, params):
    p = params
    rowb = lambda b: b.reshape(1, -1)

    idx_all = jnp.concatenate([nbr_fea_idx1, nbr_fea_idx2])
    ep = ((2 * E + _NW * _GCH - 1) // (_NW * _GCH)) * (_NW * _GCH)
    idx_all = jnp.pad(idx_all, (0, ep - 2 * E))

    zeros_nm = jnp.zeros((N, M), jnp.float32)
    nn_col = num_nbrs.reshape(N, 1)
    cidx_col = crystal_atom_idx.reshape(N, 1)

    atom = _embed(atom_fea, p["node_W"].T, rowb(p["node_b"]), 2000)
    nbr = _embed(nbr_fea, p["edge_W"].T, rowb(p["edge_b"]), 2000)

    eks_parts = None
    nconv = len(p["convs"])
    for li, c in enumerate(p["convs"]):
        gath = _sc_gather(atom, idx_all)
        e_w1t = c["eW1"].T
        ek, nbr = _edge_mlp(
            gath[:E], gath[E:2 * E], nbr,
            e_w1t[:M], e_w1t[M:2 * M], e_w1t[2 * M:], rowb(c["eb1"]),
            c["eW2"].T, rowb(c["eb2"]), c["eW3"].T, rowb(c["eb3"]), 2000)
        rho_parts = _sc_scatter(ek, nbr_fea_idx1, zeros_nm)
        v_w1t = c["vW1"].T
        atom = _node_mlp(
            atom, rho_parts[:N], rho_parts[N:], nn_col,
            v_w1t[:M], v_w1t[M:], rowb(c["vb1"]),
            c["vW2"].T, rowb(c["vb2"]), c["vW3"].T, rowb(c["vb3"]),
            rowb(c["bn_g"]), rowb(c["bn_b"]))
        if li == nconv - 1:
            eks_parts = _sc_scatter(nbr, nbr_fea_idx1, zeros_nm)

    u_w1t = p["uW1"].T
    return _pool_head(
        atom, eks_parts[:N], eks_parts[N:], nn_col, cidx_col,
        u_w1t[:M], u_w1t[M:], rowb(p["ub1"]),
        p["uW2"].T, rowb(p["ub2"]),
        p["fcW"].T, rowb(p["fcb"]),
        p["fc1W"].T, rowb(p["fc1b"]),
        p["outW"].T, rowb(p["outb"]), 2000)
